# speedup vs baseline: 1.3847x; 1.3847x over previous
import functools

import jax
import jax.numpy as jnp
from jax.experimental import pallas as pl
from jax.experimental.pallas import tpu as pltpu


def _rb_kernel(S, BUF, Rb,
               p_ref, kc_ref, vc_ref, kv_ref, vv_ref,
               ko_ref, vo_ref, kext_ref, vext_ref):
    j = pl.program_id(1)

    @pl.when(j == 0)
    def _fill():
        kext_ref[pl.ds(Rb, S), :] = kv_ref[0, 0, :, :]
        vext_ref[pl.ds(Rb, S), :] = vv_ref[0, 0, :, :]
        kext_ref[pl.ds(Rb + S, Rb), :] = kv_ref[0, 0, 0:Rb, :]
        vext_ref[pl.ds(Rb + S, Rb), :] = vv_ref[0, 0, 0:Rb, :]

    p = p_ref[0]
    base = (j * Rb - p) % BUF
    v0 = (base + Rb) % BUF - Rb
    v0 = jnp.clip(v0, -Rb, S)

    rows = jax.lax.broadcasted_iota(jnp.int32, (Rb, 1), 0)
    idx = base + rows
    idx = jnp.where(idx >= BUF, idx - BUF, idx)
    mask = idx < S

    slab_k = kext_ref[pl.ds(Rb + v0, Rb), :]
    slab_v = vext_ref[pl.ds(Rb + v0, Rb), :]
    ko_ref[0, 0, :, :] = jnp.where(mask, slab_k, kc_ref[0, 0, :, :])
    vo_ref[0, 0, :, :] = jnp.where(mask, slab_v, vc_ref[0, 0, :, :])


@jax.jit
def kernel(k_cache, v_cache, k_val, v_val, input_pos):
    B, H, BUF, D = k_cache.shape
    S = k_val.shape[2]
    Rb = 512

    p = jnp.asarray(input_pos, jnp.int32).reshape((1,)) % BUF

    cache_spec = pl.BlockSpec((1, 1, Rb, D), lambda h, j, p_ref: (0, h, j, 0))
    val_spec = pl.BlockSpec((1, 1, S, D), lambda h, j, p_ref: (0, h, 0, 0))
    out_spec = pl.BlockSpec((1, 1, Rb, D), lambda h, j, p_ref: (0, h, j, 0))

    grid_spec = pltpu.PrefetchScalarGridSpec(
        num_scalar_prefetch=1,
        grid=(H, BUF // Rb),
        in_specs=[cache_spec, cache_spec, val_spec, val_spec],
        out_specs=[out_spec, out_spec],
        scratch_shapes=[
            pltpu.VMEM((S + 2 * Rb, D), jnp.float32),
            pltpu.VMEM((S + 2 * Rb, D), jnp.float32),
        ],
    )

    k_new, v_new = pl.pallas_call(
        functools.partial(_rb_kernel, S, BUF, Rb),
        grid_spec=grid_spec,
        out_shape=[
            jax.ShapeDtypeStruct(k_cache.shape, k_cache.dtype),
            jax.ShapeDtypeStruct(v_cache.shape, v_cache.dtype),
        ],
        compiler_params=pltpu.CompilerParams(
            dimension_semantics=("arbitrary", "arbitrary"),
        ),
    )(p, k_cache, v_cache, k_val, v_val)
    return (k_new, v_new)


# speedup vs baseline: 1.9152x; 1.3831x over previous
import functools

import jax
import jax.numpy as jnp
from jax.experimental import pallas as pl
from jax.experimental.pallas import tpu as pltpu


def _rb_kernel(S, BUF, Rb,
               p_ref, kc_ref, vc_ref, kv_ref, vv_ref,
               ko_ref, vo_ref, kext_ref, vext_ref):
    j = pl.program_id(1)

    @pl.when(j == 0)
    def _fill():
        kext_ref[pl.ds(Rb, S), :] = kv_ref[0, 0, :, :]
        vext_ref[pl.ds(Rb, S), :] = vv_ref[0, 0, :, :]

    p = p_ref[0]
    base = (j * Rb - p) % BUF
    v0 = (base + Rb) % BUF - Rb
    v0 = jnp.clip(v0, -Rb, S)

    rows = jax.lax.broadcasted_iota(jnp.int32, (Rb, 1), 0)
    idx = base + rows
    idx = jnp.where(idx >= BUF, idx - BUF, idx)
    mask = idx < S

    slab_k = kext_ref[pl.ds(Rb + v0, Rb), :]
    slab_v = vext_ref[pl.ds(Rb + v0, Rb), :]
    ko_ref[0, 0, :, :] = jnp.where(mask, slab_k, kc_ref[0, 0, :, :])
    vo_ref[0, 0, :, :] = jnp.where(mask, slab_v, vc_ref[0, 0, :, :])


@jax.jit
def kernel(k_cache, v_cache, k_val, v_val, input_pos):
    B, H, BUF, D = k_cache.shape
    S = k_val.shape[2]
    Rb = 1024

    p = jnp.asarray(input_pos, jnp.int32).reshape((1,)) % BUF

    cache_spec = pl.BlockSpec((1, 1, Rb, D), lambda h, j, p_ref: (0, h, j, 0))
    val_spec = pl.BlockSpec((1, 1, S, D), lambda h, j, p_ref: (0, h, 0, 0))
    out_spec = pl.BlockSpec((1, 1, Rb, D), lambda h, j, p_ref: (0, h, j, 0))

    grid_spec = pltpu.PrefetchScalarGridSpec(
        num_scalar_prefetch=1,
        grid=(H, BUF // Rb),
        in_specs=[cache_spec, cache_spec, val_spec, val_spec],
        out_specs=[out_spec, out_spec],
        scratch_shapes=[
            pltpu.VMEM((S + 2 * Rb, D), jnp.float32),
            pltpu.VMEM((S + 2 * Rb, D), jnp.float32),
        ],
    )

    k_new, v_new = pl.pallas_call(
        functools.partial(_rb_kernel, S, BUF, Rb),
        grid_spec=grid_spec,
        out_shape=[
            jax.ShapeDtypeStruct(k_cache.shape, k_cache.dtype),
            jax.ShapeDtypeStruct(v_cache.shape, v_cache.dtype),
        ],
        compiler_params=pltpu.CompilerParams(
            dimension_semantics=("arbitrary", "arbitrary"),
        ),
    )(p, k_cache, v_cache, k_val, v_val)
    return (k_new, v_new)


# speedup vs baseline: 2.5868x; 1.3507x over previous
import functools

import jax
import jax.numpy as jnp
from jax.experimental import pallas as pl
from jax.experimental.pallas import tpu as pltpu


def _rb_kernel(S, BUF, Rb,
               p_ref, kc_ref, vc_ref, kv_ref, vv_ref,
               ko_ref, vo_ref, kext_ref, vext_ref):
    j = pl.program_id(1)

    @pl.when(j == 0)
    def _fill():
        kext_ref[pl.ds(Rb, S), :] = kv_ref[0, 0, :, :]
        vext_ref[pl.ds(Rb, S), :] = vv_ref[0, 0, :, :]

    p = p_ref[0]
    base = (j * Rb - p) % BUF
    v0 = (base + Rb) % BUF - Rb
    v0 = jnp.clip(v0, -Rb, S)

    rows = jax.lax.broadcasted_iota(jnp.int32, (Rb, 1), 0)
    idx = base + rows
    idx = jnp.where(idx >= BUF, idx - BUF, idx)
    mask = idx < S

    slab_k = kext_ref[pl.ds(Rb + v0, Rb), :]
    slab_v = vext_ref[pl.ds(Rb + v0, Rb), :]
    ko_ref[0, 0, :, :] = jnp.where(mask, slab_k, kc_ref[0, 0, :, :])
    vo_ref[0, 0, :, :] = jnp.where(mask, slab_v, vc_ref[0, 0, :, :])


@jax.jit
def kernel(k_cache, v_cache, k_val, v_val, input_pos):
    B, H, BUF, D = k_cache.shape
    S = k_val.shape[2]
    Rb = 2048

    p = jnp.asarray(input_pos, jnp.int32).reshape((1,)) % BUF

    cache_spec = pl.BlockSpec((1, 1, Rb, D), lambda h, j, p_ref: (0, h, j, 0))
    val_spec = pl.BlockSpec((1, 1, S, D), lambda h, j, p_ref: (0, h, 0, 0))
    out_spec = pl.BlockSpec((1, 1, Rb, D), lambda h, j, p_ref: (0, h, j, 0))

    grid_spec = pltpu.PrefetchScalarGridSpec(
        num_scalar_prefetch=1,
        grid=(H, BUF // Rb),
        in_specs=[cache_spec, cache_spec, val_spec, val_spec],
        out_specs=[out_spec, out_spec],
        scratch_shapes=[
            pltpu.VMEM((S + 2 * Rb, D), jnp.float32),
            pltpu.VMEM((S + 2 * Rb, D), jnp.float32),
        ],
    )

    k_new, v_new = pl.pallas_call(
        functools.partial(_rb_kernel, S, BUF, Rb),
        grid_spec=grid_spec,
        out_shape=[
            jax.ShapeDtypeStruct(k_cache.shape, k_cache.dtype),
            jax.ShapeDtypeStruct(v_cache.shape, v_cache.dtype),
        ],
        compiler_params=pltpu.CompilerParams(
            dimension_semantics=("arbitrary", "arbitrary"),
        ),
    )(p, k_cache, v_cache, k_val, v_val)
    return (k_new, v_new)


# speedup vs baseline: 2.9850x; 1.1539x over previous
import functools

import jax
import jax.numpy as jnp
from jax.experimental import pallas as pl
from jax.experimental.pallas import tpu as pltpu


def _rb_kernel(S, BUF, Rb,
               p_ref, kc_ref, vc_ref, kv_ref, vv_ref,
               ko_ref, vo_ref, kext_ref, vext_ref):
    j = pl.program_id(1)

    @pl.when(j == 0)
    def _fill():
        kext_ref[pl.ds(Rb, S), :] = kv_ref[0, 0, :, :]
        vext_ref[pl.ds(Rb, S), :] = vv_ref[0, 0, :, :]

    p = p_ref[0]
    base = (j * Rb - p) % BUF
    v0 = (base + Rb) % BUF - Rb
    v0 = jnp.clip(v0, -Rb, S)

    rows = jax.lax.broadcasted_iota(jnp.int32, (Rb, 1), 0)
    idx = base + rows
    idx = jnp.where(idx >= BUF, idx - BUF, idx)
    mask = idx < S

    slab_k = kext_ref[pl.ds(Rb + v0, Rb), :]
    slab_v = vext_ref[pl.ds(Rb + v0, Rb), :]
    ko_ref[0, 0, :, :] = jnp.where(mask, slab_k, kc_ref[0, 0, :, :])
    vo_ref[0, 0, :, :] = jnp.where(mask, slab_v, vc_ref[0, 0, :, :])


@jax.jit
def kernel(k_cache, v_cache, k_val, v_val, input_pos):
    B, H, BUF, D = k_cache.shape
    S = k_val.shape[2]
    Rb = 4096

    p = jnp.asarray(input_pos, jnp.int32).reshape((1,)) % BUF

    cache_spec = pl.BlockSpec((1, 1, Rb, D), lambda h, j, p_ref: (0, h, j, 0))
    val_spec = pl.BlockSpec((1, 1, S, D), lambda h, j, p_ref: (0, h, 0, 0))
    out_spec = pl.BlockSpec((1, 1, Rb, D), lambda h, j, p_ref: (0, h, j, 0))

    grid_spec = pltpu.PrefetchScalarGridSpec(
        num_scalar_prefetch=1,
        grid=(H, BUF // Rb),
        in_specs=[cache_spec, cache_spec, val_spec, val_spec],
        out_specs=[out_spec, out_spec],
        scratch_shapes=[
            pltpu.VMEM((S + 2 * Rb, D), jnp.float32),
            pltpu.VMEM((S + 2 * Rb, D), jnp.float32),
        ],
    )

    k_new, v_new = pl.pallas_call(
        functools.partial(_rb_kernel, S, BUF, Rb),
        grid_spec=grid_spec,
        out_shape=[
            jax.ShapeDtypeStruct(k_cache.shape, k_cache.dtype),
            jax.ShapeDtypeStruct(v_cache.shape, v_cache.dtype),
        ],
        compiler_params=pltpu.CompilerParams(
            dimension_semantics=("arbitrary", "arbitrary"),
        ),
    )(p, k_cache, v_cache, k_val, v_val)
    return (k_new, v_new)


# speedup vs baseline: 3.1729x; 1.0630x over previous
import functools

import jax
import jax.numpy as jnp
from jax.experimental import pallas as pl
from jax.experimental.pallas import tpu as pltpu


def _rb_kernel(S, BUF, Rb,
               p_ref, kc_ref, vc_ref, kv_ref, vv_ref,
               ko_ref, vo_ref, kext_ref, vext_ref):
    j = pl.program_id(1)

    @pl.when(j == 0)
    def _fill():
        kext_ref[pl.ds(Rb, S), :] = kv_ref[0, 0, :, :]
        vext_ref[pl.ds(Rb, S), :] = vv_ref[0, 0, :, :]

    p = p_ref[0]
    base = (j * Rb - p) % BUF
    v0 = (base + Rb) % BUF - Rb
    v0 = jnp.clip(v0, -Rb, S)

    rows = jax.lax.broadcasted_iota(jnp.int32, (Rb, 1), 0)
    idx = base + rows
    idx = jnp.where(idx >= BUF, idx - BUF, idx)
    mask = idx < S

    slab_k = kext_ref[pl.ds(Rb + v0, Rb), :]
    slab_v = vext_ref[pl.ds(Rb + v0, Rb), :]
    ko_ref[0, 0, :, :] = jnp.where(mask, slab_k, kc_ref[0, 0, :, :])
    vo_ref[0, 0, :, :] = jnp.where(mask, slab_v, vc_ref[0, 0, :, :])


@jax.jit
def kernel(k_cache, v_cache, k_val, v_val, input_pos):
    B, H, BUF, D = k_cache.shape
    S = k_val.shape[2]
    Rb = 8192

    p = jnp.asarray(input_pos, jnp.int32).reshape((1,)) % BUF

    cache_spec = pl.BlockSpec((1, 1, Rb, D), lambda h, j, p_ref: (0, h, j, 0))
    val_spec = pl.BlockSpec((1, 1, S, D), lambda h, j, p_ref: (0, h, 0, 0))
    out_spec = pl.BlockSpec((1, 1, Rb, D), lambda h, j, p_ref: (0, h, j, 0))

    grid_spec = pltpu.PrefetchScalarGridSpec(
        num_scalar_prefetch=1,
        grid=(H, BUF // Rb),
        in_specs=[cache_spec, cache_spec, val_spec, val_spec],
        out_specs=[out_spec, out_spec],
        scratch_shapes=[
            pltpu.VMEM((S + 2 * Rb, D), jnp.float32),
            pltpu.VMEM((S + 2 * Rb, D), jnp.float32),
        ],
    )

    k_new, v_new = pl.pallas_call(
        functools.partial(_rb_kernel, S, BUF, Rb),
        grid_spec=grid_spec,
        out_shape=[
            jax.ShapeDtypeStruct(k_cache.shape, k_cache.dtype),
            jax.ShapeDtypeStruct(v_cache.shape, v_cache.dtype),
        ],
        compiler_params=pltpu.CompilerParams(
            dimension_semantics=("arbitrary", "arbitrary"),
        ),
    )(p, k_cache, v_cache, k_val, v_val)
    return (k_new, v_new)
